# baseline (speedup 1.0000x reference)
import functools

import jax
import jax.numpy as jnp
from jax import lax
from jax.experimental import pallas as pl
from jax.experimental.pallas import tpu as pltpu
from jax.experimental.pallas import tpu_sc as plsc

DIS = 26
CONT = 13
FEAT = DIS + CONT
VOCAB = 100000
VPAD = 100096
BATCH = 16384
LANES = 16
NW = 32
RPW = BATCH // NW
NSL = RPW // LANES
GS = 4
NG = NSL // GS
CH = GS * DIS * LANES


def _sc_body(xt_hbm, tab_hbm, out_hbm, xbuf, idxbuf, gbuf, obuf, sem):
    nc = plsc.get_sparse_core_info().num_cores
    wid = lax.axis_index("s") * nc + lax.axis_index("c")
    base = wid * RPW

    for i in range(FEAT):
        pltpu.async_copy(
            xt_hbm.at[pl.ds(i * BATCH + base, RPW)],
            xbuf.at[pl.ds(i * RPW, RPW)], sem)
    for i in range(FEAT):
        pltpu.make_async_copy(
            xt_hbm.at[pl.ds(i * BATCH + base, RPW)],
            xbuf.at[pl.ds(i * RPW, RPW)], sem).wait()

    def idx_slice(s, _):
        o = s * LANES
        ib = s * (DIS * LANES)
        for i in range(DIS):
            iv = xbuf[pl.ds(i * RPW + o, LANES)].astype(jnp.int32) + i * VPAD
            idxbuf[pl.ds(ib + i * LANES, LANES)] = iv
        acc = xbuf[pl.ds(DIS * RPW + o, LANES)]
        for k in range(1, CONT):
            acc = acc + xbuf[pl.ds((DIS + k) * RPW + o, LANES)]
        obuf[pl.ds(o, LANES)] = acc

        @pl.when(s % GS == GS - 1)
        def _fire():
            g = s // GS
            pltpu.async_copy(
                tab_hbm.at[idxbuf.at[pl.ds(g * CH, CH)]],
                gbuf.at[pl.ds(g * CH, CH)], sem)

        return 0

    lax.fori_loop(0, NSL, idx_slice, 0)

    for g in range(NG):
        pltpu.make_async_copy(
            tab_hbm.at[idxbuf.at[pl.ds(g * CH, CH)]],
            gbuf.at[pl.ds(g * CH, CH)], sem).wait()

        def red_slice(s, _):
            o = s * LANES
            ib = s * (DIS * LANES)
            acc = obuf[pl.ds(o, LANES)]
            for i in range(DIS):
                acc = acc + gbuf[pl.ds(ib + i * LANES, LANES)]
            obuf[pl.ds(o, LANES)] = 1.0 / (1.0 + jnp.exp(-acc))
            return 0

        lax.fori_loop(g * GS, (g + 1) * GS, red_slice, 0)

    pltpu.sync_copy(obuf, out_hbm.at[pl.ds(base, RPW)])


def kernel(X, tables):
    xt = X.T.reshape(FEAT * BATCH)
    tab = jnp.pad(
        tables.reshape(DIS, VOCAB), ((0, 0), (0, VPAD - VOCAB))
    ).reshape(DIS * VPAD)
    mesh = plsc.VectorSubcoreMesh(core_axis_name="c", subcore_axis_name="s")
    run = functools.partial(
        pl.kernel,
        mesh=mesh,
        out_type=jax.ShapeDtypeStruct((BATCH,), jnp.float32),
        compiler_params=pltpu.CompilerParams(needs_layout_passes=False),
        scratch_types=[
            pltpu.VMEM((FEAT * RPW,), jnp.float32),
            pltpu.VMEM((DIS * RPW,), jnp.int32),
            pltpu.VMEM((DIS * RPW,), jnp.float32),
            pltpu.VMEM((RPW,), jnp.float32),
            pltpu.SemaphoreType.DMA,
        ],
    )(_sc_body)
    out = run(xt, tab)
    return out.reshape(BATCH, 1)


# speedup vs baseline: 1.0029x; 1.0029x over previous
import functools

import jax
import jax.numpy as jnp
from jax import lax
from jax.experimental import pallas as pl
from jax.experimental.pallas import tpu as pltpu
from jax.experimental.pallas import tpu_sc as plsc

DIS = 26
CONT = 13
FEAT = DIS + CONT
VOCAB = 100000
VPAD = 100096
BATCH = 16384
LANES = 16
NW = 32
RPW = BATCH // NW
NSL = RPW // LANES
GROUPS = (10, 10, 10, 2)
GSTART = (0, 10, 20, 30)
GW = DIS * LANES


def _sc_body(xt_hbm, tab_hbm, out_hbm, xbuf, idxbuf, gbuf, obuf, sem):
    nc = plsc.get_sparse_core_info().num_cores
    wid = lax.axis_index("s") * nc + lax.axis_index("c")
    base = wid * RPW

    for i in range(FEAT):
        pltpu.async_copy(
            xt_hbm.at[pl.ds(i * BATCH + base, RPW)],
            xbuf.at[pl.ds(i * RPW, RPW)], sem)
    for i in range(FEAT):
        pltpu.make_async_copy(
            xt_hbm.at[pl.ds(i * BATCH + base, RPW)],
            xbuf.at[pl.ds(i * RPW, RPW)], sem).wait()

    def idx_slice(s, _):
        o = s * LANES
        ib = s * (DIS * LANES)
        for i in range(DIS):
            iv = xbuf[pl.ds(i * RPW + o, LANES)].astype(jnp.int32) + i * VPAD
            idxbuf[pl.ds(ib + i * LANES, LANES)] = iv
        acc = xbuf[pl.ds(DIS * RPW + o, LANES)]
        for k in range(1, CONT):
            acc = acc + xbuf[pl.ds((DIS + k) * RPW + o, LANES)]
        obuf[pl.ds(o, LANES)] = acc

        for g, (st, ln) in enumerate(zip(GSTART, GROUPS)):
            @pl.when(s == st + ln - 1)
            def _fire(st=st, ln=ln):
                pltpu.async_copy(
                    tab_hbm.at[idxbuf.at[pl.ds(st * GW, ln * GW)]],
                    gbuf.at[pl.ds(st * GW, ln * GW)], sem)

        return 0

    lax.fori_loop(0, NSL, idx_slice, 0)

    for st, ln in zip(GSTART, GROUPS):
        pltpu.make_async_copy(
            tab_hbm.at[idxbuf.at[pl.ds(st * GW, ln * GW)]],
            gbuf.at[pl.ds(st * GW, ln * GW)], sem).wait()

        def red_slice(s, _):
            o = s * LANES
            ib = s * (DIS * LANES)
            acc = obuf[pl.ds(o, LANES)]
            for i in range(DIS):
                acc = acc + gbuf[pl.ds(ib + i * LANES, LANES)]
            obuf[pl.ds(o, LANES)] = 1.0 / (1.0 + jnp.exp(-acc))
            return 0

        lax.fori_loop(st, st + ln, red_slice, 0)

    pltpu.sync_copy(obuf, out_hbm.at[pl.ds(base, RPW)])


def kernel(X, tables):
    xt = X.T.reshape(FEAT * BATCH)
    tab = jnp.pad(
        tables.reshape(DIS, VOCAB), ((0, 0), (0, VPAD - VOCAB))
    ).reshape(DIS * VPAD)
    mesh = plsc.VectorSubcoreMesh(core_axis_name="c", subcore_axis_name="s")
    run = functools.partial(
        pl.kernel,
        mesh=mesh,
        out_type=jax.ShapeDtypeStruct((BATCH,), jnp.float32),
        compiler_params=pltpu.CompilerParams(needs_layout_passes=False),
        scratch_types=[
            pltpu.VMEM((FEAT * RPW,), jnp.float32),
            pltpu.VMEM((DIS * RPW,), jnp.int32),
            pltpu.VMEM((DIS * RPW,), jnp.float32),
            pltpu.VMEM((RPW,), jnp.float32),
            pltpu.SemaphoreType.DMA,
        ],
    )(_sc_body)
    out = run(xt, tab)
    return out.reshape(BATCH, 1)


# speedup vs baseline: 1.0060x; 1.0031x over previous
import functools

import jax
import jax.numpy as jnp
from jax import lax
from jax.experimental import pallas as pl
from jax.experimental.pallas import tpu as pltpu
from jax.experimental.pallas import tpu_sc as plsc

DIS = 26
CONT = 13
FEAT = DIS + CONT
VOCAB = 100000
VPAD = 100096
BATCH = 16384
LANES = 16
NW = 32
RPW = BATCH // NW
NSL = RPW // LANES
GS = 8
NG = NSL // GS
CH = GS * DIS * LANES


def _sc_body(xt_hbm, tab_hbm, out_hbm, xbuf, idxbuf, gbuf, obuf, sem):
    nc = plsc.get_sparse_core_info().num_cores
    wid = lax.axis_index("s") * nc + lax.axis_index("c")
    base = wid * RPW

    for i in range(FEAT):
        pltpu.async_copy(
            xt_hbm.at[pl.ds(i * BATCH + base, RPW)],
            xbuf.at[pl.ds(i * RPW, RPW)], sem)
    for i in range(FEAT):
        pltpu.make_async_copy(
            xt_hbm.at[pl.ds(i * BATCH + base, RPW)],
            xbuf.at[pl.ds(i * RPW, RPW)], sem).wait()

    def idx_slice(s, _):
        o = s * LANES
        ib = s * (DIS * LANES)
        for i in range(DIS):
            iv = xbuf[pl.ds(i * RPW + o, LANES)].astype(jnp.int32) + i * VPAD
            idxbuf[pl.ds(ib + i * LANES, LANES)] = iv
        acc = xbuf[pl.ds(DIS * RPW + o, LANES)]
        for k in range(1, CONT):
            acc = acc + xbuf[pl.ds((DIS + k) * RPW + o, LANES)]
        obuf[pl.ds(o, LANES)] = acc

        @pl.when(s % GS == GS - 1)
        def _fire():
            g = s // GS
            pltpu.async_copy(
                tab_hbm.at[idxbuf.at[pl.ds(g * CH, CH)]],
                gbuf.at[pl.ds(g * CH, CH)], sem)

        return 0

    lax.fori_loop(0, NSL, idx_slice, 0)

    for g in range(NG):
        pltpu.make_async_copy(
            tab_hbm.at[idxbuf.at[pl.ds(g * CH, CH)]],
            gbuf.at[pl.ds(g * CH, CH)], sem).wait()

        def red_slice(s, _):
            o = s * LANES
            ib = s * (DIS * LANES)
            acc = obuf[pl.ds(o, LANES)]
            for i in range(DIS):
                acc = acc + gbuf[pl.ds(ib + i * LANES, LANES)]
            obuf[pl.ds(o, LANES)] = 1.0 / (1.0 + jnp.exp(-acc))
            return 0

        lax.fori_loop(g * GS, (g + 1) * GS, red_slice, 0)

    pltpu.sync_copy(obuf, out_hbm.at[pl.ds(base, RPW)])


def kernel(X, tables):
    xt = X.T.reshape(FEAT * BATCH)
    tab = jnp.pad(
        tables.reshape(DIS, VOCAB), ((0, 0), (0, VPAD - VOCAB))
    ).reshape(DIS * VPAD)
    mesh = plsc.VectorSubcoreMesh(core_axis_name="c", subcore_axis_name="s")
    run = functools.partial(
        pl.kernel,
        mesh=mesh,
        out_type=jax.ShapeDtypeStruct((BATCH,), jnp.float32),
        compiler_params=pltpu.CompilerParams(needs_layout_passes=False),
        scratch_types=[
            pltpu.VMEM((FEAT * RPW,), jnp.float32),
            pltpu.VMEM((DIS * RPW,), jnp.int32),
            pltpu.VMEM((DIS * RPW,), jnp.float32),
            pltpu.VMEM((RPW,), jnp.float32),
            pltpu.SemaphoreType.DMA,
        ],
    )(_sc_body)
    out = run(xt, tab)
    return out.reshape(BATCH, 1)


# speedup vs baseline: 1.0078x; 1.0018x over previous
import functools

import jax
import jax.numpy as jnp
from jax import lax
from jax.experimental import pallas as pl
from jax.experimental.pallas import tpu as pltpu
from jax.experimental.pallas import tpu_sc as plsc

DIS = 26
CONT = 13
FEAT = DIS + CONT
VOCAB = 100000
VPAD = 100096
BATCH = 16384
LANES = 16
NW = 32
RPW = BATCH // NW
NSL = RPW // LANES
GS = 8
NG = NSL // GS
CH = GS * DIS * LANES


def _sc_body(xt_hbm, tab_hbm, out_hbm, xbuf, idxbuf, gbuf, obuf, sem):
    nc = plsc.get_sparse_core_info().num_cores
    wid = lax.axis_index("s") * nc + lax.axis_index("c")
    base = wid * RPW

    for i in range(FEAT):
        pltpu.async_copy(
            xt_hbm.at[pl.ds(i * BATCH + base, RPW)],
            xbuf.at[pl.ds(i * RPW, RPW)], sem)
    for i in range(FEAT):
        pltpu.make_async_copy(
            xt_hbm.at[pl.ds(i * BATCH + base, RPW)],
            xbuf.at[pl.ds(i * RPW, RPW)], sem).wait()

    def idx_slice(s, _):
        o = s * LANES
        ib = s * (DIS * LANES)
        for i in range(DIS):
            iv = xbuf[pl.ds(i * RPW + o, LANES)].astype(jnp.int32) + i * VPAD
            idxbuf[pl.ds(ib + i * LANES, LANES)] = iv

        @pl.when(s % GS == GS - 1)
        def _fire():
            g = s // GS
            pltpu.async_copy(
                tab_hbm.at[idxbuf.at[pl.ds(g * CH, CH)]],
                gbuf.at[pl.ds(g * CH, CH)], sem)

        return 0

    lax.fori_loop(0, NSL, idx_slice, 0)

    def cont_slice(s, _):
        o = s * LANES
        acc = xbuf[pl.ds(DIS * RPW + o, LANES)]
        for k in range(1, CONT):
            acc = acc + xbuf[pl.ds((DIS + k) * RPW + o, LANES)]
        obuf[pl.ds(o, LANES)] = acc
        return 0

    lax.fori_loop(0, NSL, cont_slice, 0)

    for g in range(NG):
        pltpu.make_async_copy(
            tab_hbm.at[idxbuf.at[pl.ds(g * CH, CH)]],
            gbuf.at[pl.ds(g * CH, CH)], sem).wait()

        def red_slice(s, _):
            o = s * LANES
            ib = s * (DIS * LANES)
            acc = obuf[pl.ds(o, LANES)]
            for i in range(DIS):
                acc = acc + gbuf[pl.ds(ib + i * LANES, LANES)]
            obuf[pl.ds(o, LANES)] = 1.0 / (1.0 + jnp.exp(-acc))
            return 0

        lax.fori_loop(g * GS, (g + 1) * GS, red_slice, 0)

    pltpu.sync_copy(obuf, out_hbm.at[pl.ds(base, RPW)])


def kernel(X, tables):
    xt = X.T.reshape(FEAT * BATCH)
    tab = jnp.pad(
        tables.reshape(DIS, VOCAB), ((0, 0), (0, VPAD - VOCAB))
    ).reshape(DIS * VPAD)
    mesh = plsc.VectorSubcoreMesh(core_axis_name="c", subcore_axis_name="s")
    run = functools.partial(
        pl.kernel,
        mesh=mesh,
        out_type=jax.ShapeDtypeStruct((BATCH,), jnp.float32),
        compiler_params=pltpu.CompilerParams(needs_layout_passes=False),
        scratch_types=[
            pltpu.VMEM((FEAT * RPW,), jnp.float32),
            pltpu.VMEM((DIS * RPW,), jnp.int32),
            pltpu.VMEM((DIS * RPW,), jnp.float32),
            pltpu.VMEM((RPW,), jnp.float32),
            pltpu.SemaphoreType.DMA,
        ],
    )(_sc_body)
    out = run(xt, tab)
    return out.reshape(BATCH, 1)
